# Initial kernel scaffold; baseline (speedup 1.0000x reference)
import jax
import jax.numpy as jnp
from jax.experimental import pallas as pl


def kernel(token_ids, weight):
    raise NotImplementedError("write your pallas kernel here")


# speedup vs baseline: 1.8317x; 1.8317x over previous
import functools

import jax
import jax.numpy as jnp
from jax import lax
from jax.experimental import pallas as pl
from jax.experimental.pallas import tpu as pltpu
from jax.experimental.pallas import tpu_sc as plsc

_NUM_ROWS = 1000000
_DIM = 64
_BATCH = 16384 * 50
_NUM_WORKERS = 32
_B_PER_W = _BATCH // _NUM_WORKERS
_CHUNK = 512
_NCHUNKS = _B_PER_W // _CHUNK


@functools.partial(
    pl.kernel,
    mesh=plsc.VectorSubcoreMesh(core_axis_name="c", subcore_axis_name="s"),
    out_type=jax.ShapeDtypeStruct((_BATCH, _DIM), jnp.float32),
    scratch_types=[
        pltpu.VMEM((_B_PER_W,), jnp.int32),
        pltpu.VMEM((_CHUNK, _DIM), jnp.float32),
        pltpu.SemaphoreType.DMA,
    ],
    compiler_params=pltpu.CompilerParams(use_tc_tiling_on_sc=False),
)
def _embed_gather(idx_hbm, table_hbm, out_hbm, idx_v, rows_v, sem):
    wid = lax.axis_index("s") * 2 + lax.axis_index("c")
    base = wid * _B_PER_W
    pltpu.sync_copy(idx_hbm.at[pl.ds(base, _B_PER_W)], idx_v)

    def body(i, carry):
        off = pl.multiple_of(i * _CHUNK, 8)
        pltpu.async_copy(
            table_hbm.at[idx_v.at[pl.ds(off, _CHUNK)]], rows_v, sem
        ).wait()
        pltpu.sync_copy(rows_v, out_hbm.at[pl.ds(base + off, _CHUNK)])
        return carry

    lax.fori_loop(0, _NCHUNKS, body, 0)


def kernel(token_ids, weight):
    idx = jnp.reshape(token_ids.astype(jnp.int32), (_BATCH,))
    out = _embed_gather(idx, weight)
    return jnp.reshape(out, (*token_ids.shape, _DIM))


# speedup vs baseline: 1.8683x; 1.0199x over previous
import functools

import jax
import jax.numpy as jnp
from jax import lax
from jax.experimental import pallas as pl
from jax.experimental.pallas import tpu as pltpu
from jax.experimental.pallas import tpu_sc as plsc

_NUM_ROWS = 1000000
_DIM = 64
_BATCH = 16384 * 50
_NUM_WORKERS = 32
_B_PER_W = _BATCH // _NUM_WORKERS
_NBUF = 4
_CHUNK = 320
_NCHUNKS = _B_PER_W // _CHUNK
_NOUTER = _NCHUNKS // _NBUF


@functools.partial(
    pl.kernel,
    mesh=plsc.VectorSubcoreMesh(core_axis_name="c", subcore_axis_name="s"),
    out_type=jax.ShapeDtypeStruct((_BATCH, _DIM), jnp.float32),
    scratch_types=[
        pltpu.VMEM((_B_PER_W,), jnp.int32),
        pltpu.VMEM((_NBUF, _CHUNK, _DIM), jnp.float32),
    ] + [pltpu.SemaphoreType.DMA] * (2 * _NBUF),
    compiler_params=pltpu.CompilerParams(use_tc_tiling_on_sc=False),
)
def _embed_gather(idx_hbm, table_hbm, out_hbm, idx_v, rows_v, *sems):
    gsems = sems[:_NBUF]
    ssems = sems[_NBUF:]
    wid = lax.axis_index("s") * 2 + lax.axis_index("c")
    base = wid * _B_PER_W
    pltpu.sync_copy(idx_hbm.at[pl.ds(base, _B_PER_W)], idx_v)

    def start_gather(chunk, b):
        off = pl.multiple_of(chunk * _CHUNK, 8)
        pltpu.async_copy(
            table_hbm.at[idx_v.at[pl.ds(off, _CHUNK)]], rows_v.at[b], gsems[b]
        )

    def wait_gather(b):
        pltpu.make_async_copy(
            table_hbm.at[pl.ds(0, _CHUNK)], rows_v.at[b], gsems[b]
        ).wait()

    def start_store(chunk, b):
        off = pl.multiple_of(base + chunk * _CHUNK, 8)
        pltpu.async_copy(rows_v.at[b], out_hbm.at[pl.ds(off, _CHUNK)], ssems[b])

    def wait_store(b):
        pltpu.make_async_copy(
            rows_v.at[b], out_hbm.at[pl.ds(0, _CHUNK)], ssems[b]
        ).wait()

    for b in range(_NBUF):
        start_gather(b, b)

    def outer(g, carry):
        cbase = g * _NBUF
        for b in range(_NBUF):
            wait_gather(b)
            start_store(cbase + b, b)
        for b in range(_NBUF):

            @pl.when(g < _NOUTER - 1)
            def _():
                wait_store(b)
                start_gather(cbase + _NBUF + b, b)

        return carry

    lax.fori_loop(0, _NOUTER, outer, 0)
    for b in range(_NBUF):
        wait_store(b)


def kernel(token_ids, weight):
    idx = jnp.reshape(token_ids.astype(jnp.int32), (_BATCH,))
    out = _embed_gather(idx, weight)
    return jnp.reshape(out, (*token_ids.shape, _DIM))
